# baseline (speedup 1.0000x reference)
import functools

import jax
import jax.numpy as jnp
from jax import lax
from jax.experimental import pallas as pl
from jax.experimental.pallas import tpu as pltpu
from jax.experimental.pallas import tpu_sc as plsc

NUM_CLASSES = 100000
HIDDEN_SIZE = 128
BATCH = 16384

_INFO = plsc.get_sparse_core_info()
_NC = _INFO.num_cores
_NS = _INFO.num_subcores
_NW = _NC * _NS
_B_PER_W = BATCH // _NW
_CH = 64
_NCH = _B_PER_W // _CH


@functools.partial(
    pl.kernel,
    out_type=jax.ShapeDtypeStruct((BATCH, HIDDEN_SIZE), jnp.float32),
    mesh=plsc.VectorSubcoreMesh(core_axis_name="c", subcore_axis_name="s"),
    scratch_types=[
        pltpu.VMEM((_NCH, _CH), jnp.int32),
        pltpu.VMEM((_B_PER_W, HIDDEN_SIZE), jnp.float32),
        pltpu.SemaphoreType.DMA,
        pltpu.SemaphoreType.DMA,
        pltpu.SemaphoreType.DMA,
    ],
)
def _gather_kernel(table_hbm, idx_hbm, out_hbm, idx_v, rows_v, i_sem, g_sem, w_sem):
    wid = lax.axis_index("s") * _NC + lax.axis_index("c")
    base = wid * _B_PER_W
    idx_copies = [
        pltpu.async_copy(idx_hbm.at[wid, c], idx_v.at[c], i_sem)
        for c in range(_NCH)
    ]
    gathers = []
    for c in range(_NCH):
        idx_copies[c].wait()
        gathers.append(
            pltpu.async_copy(
                table_hbm.at[idx_v.at[c]],
                rows_v.at[pl.ds(c * _CH, _CH)],
                g_sem,
            )
        )
    writes = []
    for c in range(_NCH):
        gathers[c].wait()
        writes.append(
            pltpu.async_copy(
                rows_v.at[pl.ds(c * _CH, _CH)],
                out_hbm.at[pl.ds(base + c * _CH, _CH)],
                w_sem,
            )
        )
    for wr in writes:
        wr.wait()


def kernel(labels, table):
    idx = labels.astype(jnp.int32).reshape(_NW, _NCH, _CH)
    return _gather_kernel(table, idx)


# speedup vs baseline: 1.0157x; 1.0157x over previous
import functools

import jax
import jax.numpy as jnp
from jax import lax
from jax.experimental import pallas as pl
from jax.experimental.pallas import tpu as pltpu
from jax.experimental.pallas import tpu_sc as plsc

NUM_CLASSES = 100000
HIDDEN_SIZE = 128
BATCH = 16384

_INFO = plsc.get_sparse_core_info()
_NC = _INFO.num_cores
_NS = _INFO.num_subcores
_NW = _NC * _NS
_B_PER_W = BATCH // _NW
_CH = 128
_NCH = _B_PER_W // _CH


@functools.partial(
    pl.kernel,
    out_type=jax.ShapeDtypeStruct((BATCH, HIDDEN_SIZE), jnp.float32),
    mesh=plsc.VectorSubcoreMesh(core_axis_name="c", subcore_axis_name="s"),
    scratch_types=[
        pltpu.VMEM((_NCH, _CH), jnp.int32),
        pltpu.VMEM((_B_PER_W, HIDDEN_SIZE), jnp.float32),
        pltpu.SemaphoreType.DMA,
        pltpu.SemaphoreType.DMA,
        pltpu.SemaphoreType.DMA,
    ],
)
def _gather_kernel(table_hbm, idx_hbm, out_hbm, idx_v, rows_v, i_sem, g_sem, w_sem):
    wid = lax.axis_index("s") * _NC + lax.axis_index("c")
    base = wid * _B_PER_W
    idx_copies = [
        pltpu.async_copy(idx_hbm.at[wid, c], idx_v.at[c], i_sem)
        for c in range(_NCH)
    ]
    gathers = []
    for c in range(_NCH):
        idx_copies[c].wait()
        gathers.append(
            pltpu.async_copy(
                table_hbm.at[idx_v.at[c]],
                rows_v.at[pl.ds(c * _CH, _CH)],
                g_sem,
            )
        )
    writes = []
    for c in range(_NCH):
        gathers[c].wait()
        writes.append(
            pltpu.async_copy(
                rows_v.at[pl.ds(c * _CH, _CH)],
                out_hbm.at[pl.ds(base + c * _CH, _CH)],
                w_sem,
            )
        )
    for wr in writes:
        wr.wait()


def kernel(labels, table):
    idx = labels.astype(jnp.int32).reshape(_NW, _NCH, _CH)
    return _gather_kernel(table, idx)
